# baseline (speedup 1.0000x reference)
import functools

import jax
import jax.numpy as jnp
from jax.experimental import pallas as pl
from jax.experimental.pallas import tpu as pltpu

SEQ_LEN = 4096
EMB = 1024
NDMA = 8
ROWS = SEQ_LEN // NDMA


def kernel(input, encoding):
    def body(inp_smem, enc_hbm, out_hbm, sems):
        start = pl.multiple_of(inp_smem[1] - SEQ_LEN, 8)
        copies = []
        for k in range(NDMA):
            copies.append(
                pltpu.make_async_copy(
                    enc_hbm.at[pl.ds(start + k * ROWS, ROWS)],
                    out_hbm.at[pl.ds(k * ROWS, ROWS)],
                    sems.at[k],
                )
            )
            copies[-1].start()
        for c in copies:
            c.wait()

    return pl.pallas_call(
        body,
        in_specs=[
            pl.BlockSpec(memory_space=pltpu.MemorySpace.SMEM),
            pl.BlockSpec(memory_space=pltpu.MemorySpace.HBM),
        ],
        out_specs=pl.BlockSpec(memory_space=pltpu.MemorySpace.HBM),
        out_shape=jax.ShapeDtypeStruct((SEQ_LEN, EMB), jnp.float32),
        scratch_shapes=[pltpu.SemaphoreType.DMA((NDMA,))],
    )(input, encoding)


# speedup vs baseline: 29.1594x; 29.1594x over previous
import jax
import jax.numpy as jnp
from jax.experimental import pallas as pl
from jax.experimental.pallas import tpu as pltpu

SEQ_LEN = 4096
EMB = 1024
CHUNK = 256
NBUF = 3
NCHUNKS = SEQ_LEN // CHUNK


def kernel(input, encoding):
    def body(inp_smem, enc_hbm, out_hbm, buf, gsems, ssems):
        start = pl.multiple_of(inp_smem[1] - SEQ_LEN, 8)
        g = [None] * NCHUNKS
        s = [None] * NCHUNKS
        for i in range(NCHUNKS):
            b = i % NBUF
            if i >= NBUF:
                s[i - NBUF].wait()
            g[i] = pltpu.make_async_copy(
                enc_hbm.at[pl.ds(start + i * CHUNK, CHUNK)], buf.at[b], gsems.at[b]
            )
            g[i].start()
            if i >= 1:
                ob = (i - 1) % NBUF
                g[i - 1].wait()
                s[i - 1] = pltpu.make_async_copy(
                    buf.at[ob], out_hbm.at[pl.ds((i - 1) * CHUNK, CHUNK)], ssems.at[ob]
                )
                s[i - 1].start()
        lb = (NCHUNKS - 1) % NBUF
        g[NCHUNKS - 1].wait()
        s[NCHUNKS - 1] = pltpu.make_async_copy(
            buf.at[lb], out_hbm.at[pl.ds((NCHUNKS - 1) * CHUNK, CHUNK)], ssems.at[lb]
        )
        s[NCHUNKS - 1].start()
        for i in range(max(0, NCHUNKS - NBUF), NCHUNKS):
            s[i].wait()

    return pl.pallas_call(
        body,
        in_specs=[
            pl.BlockSpec(memory_space=pltpu.MemorySpace.SMEM),
            pl.BlockSpec(memory_space=pltpu.MemorySpace.HBM),
        ],
        out_specs=pl.BlockSpec(memory_space=pltpu.MemorySpace.HBM),
        out_shape=jax.ShapeDtypeStruct((SEQ_LEN, EMB), jnp.float32),
        scratch_shapes=[
            pltpu.VMEM((NBUF, CHUNK, EMB), jnp.float32),
            pltpu.SemaphoreType.DMA((NBUF,)),
            pltpu.SemaphoreType.DMA((NBUF,)),
        ],
    )(input, encoding)


# speedup vs baseline: 33.7454x; 1.1573x over previous
import jax
import jax.numpy as jnp
from jax.experimental import pallas as pl
from jax.experimental.pallas import tpu as pltpu

SEQ_LEN = 4096
EMB = 1024
CHUNK = 128
NBUF = 8
DEPTH = 3
NCHUNKS = SEQ_LEN // CHUNK


def kernel(input, encoding):
    def body(inp_smem, enc_hbm, out_hbm, buf, gsems, ssems):
        start = pl.multiple_of(inp_smem[1] - SEQ_LEN, 8)
        g = [None] * NCHUNKS
        s = [None] * NCHUNKS
        for i in range(NCHUNKS + DEPTH):
            if i < NCHUNKS:
                b = i % NBUF
                if i >= NBUF:
                    s[i - NBUF].wait()
                g[i] = pltpu.make_async_copy(
                    enc_hbm.at[pl.ds(start + i * CHUNK, CHUNK)],
                    buf.at[b],
                    gsems.at[b],
                )
                g[i].start()
            j = i - DEPTH
            if 0 <= j < NCHUNKS:
                jb = j % NBUF
                g[j].wait()
                s[j] = pltpu.make_async_copy(
                    buf.at[jb], out_hbm.at[pl.ds(j * CHUNK, CHUNK)], ssems.at[jb]
                )
                s[j].start()
        for j in range(max(0, NCHUNKS - NBUF), NCHUNKS):
            s[j].wait()

    return pl.pallas_call(
        body,
        in_specs=[
            pl.BlockSpec(memory_space=pltpu.MemorySpace.SMEM),
            pl.BlockSpec(memory_space=pltpu.MemorySpace.HBM),
        ],
        out_specs=pl.BlockSpec(memory_space=pltpu.MemorySpace.HBM),
        out_shape=jax.ShapeDtypeStruct((SEQ_LEN, EMB), jnp.float32),
        scratch_shapes=[
            pltpu.VMEM((NBUF, CHUNK, EMB), jnp.float32),
            pltpu.SemaphoreType.DMA((NBUF,)),
            pltpu.SemaphoreType.DMA((NBUF,)),
        ],
    )(input, encoding)


# speedup vs baseline: 36.5544x; 1.0832x over previous
import jax
import jax.numpy as jnp
from jax.experimental import pallas as pl
from jax.experimental.pallas import tpu as pltpu

SEQ_LEN = 4096
EMB = 1024
CHUNK = 64
NBUF = 16
DEPTH = 8
NCHUNKS = SEQ_LEN // CHUNK


def kernel(input, encoding):
    def body(inp_smem, enc_hbm, out_hbm, buf, gsems, ssems):
        start = pl.multiple_of(inp_smem[1] - SEQ_LEN, 8)
        g = [None] * NCHUNKS
        s = [None] * NCHUNKS
        for i in range(NCHUNKS + DEPTH):
            if i < NCHUNKS:
                b = i % NBUF
                if i >= NBUF:
                    s[i - NBUF].wait()
                g[i] = pltpu.make_async_copy(
                    enc_hbm.at[pl.ds(start + i * CHUNK, CHUNK)],
                    buf.at[b],
                    gsems.at[b],
                )
                g[i].start()
            j = i - DEPTH
            if 0 <= j < NCHUNKS:
                jb = j % NBUF
                g[j].wait()
                s[j] = pltpu.make_async_copy(
                    buf.at[jb], out_hbm.at[pl.ds(j * CHUNK, CHUNK)], ssems.at[jb]
                )
                s[j].start()
        for j in range(max(0, NCHUNKS - NBUF), NCHUNKS):
            s[j].wait()

    return pl.pallas_call(
        body,
        in_specs=[
            pl.BlockSpec(memory_space=pltpu.MemorySpace.SMEM),
            pl.BlockSpec(memory_space=pltpu.MemorySpace.HBM),
        ],
        out_specs=pl.BlockSpec(memory_space=pltpu.MemorySpace.HBM),
        out_shape=jax.ShapeDtypeStruct((SEQ_LEN, EMB), jnp.float32),
        scratch_shapes=[
            pltpu.VMEM((NBUF, CHUNK, EMB), jnp.float32),
            pltpu.SemaphoreType.DMA((NBUF,)),
            pltpu.SemaphoreType.DMA((NBUF,)),
        ],
    )(input, encoding)
